# baseline (speedup 1.0000x reference)
import functools

import jax
import jax.numpy as jnp
from jax import lax
from jax.experimental import pallas as pl
from jax.experimental.pallas import tpu as pltpu
from jax.experimental.pallas import tpu_sc as plsc

_B = 16384
_D = 64
_S = 16

_info = plsc.get_sparse_core_info()
_NC = _info.num_cores
_NW = _info.num_cores * _info.num_subcores
_BPW = _B // _NW

_mesh = plsc.VectorSubcoreMesh(core_axis_name="c", subcore_axis_name="s")


@functools.partial(
    pl.kernel,
    mesh=_mesh,
    compiler_params=pltpu.CompilerParams(use_tc_tiling_on_sc=False),
    out_type=jax.ShapeDtypeStruct((_B, _D), jnp.float32),
    scratch_types=[
        pltpu.VMEM((_BPW,), jnp.int32),
        pltpu.VMEM((_BPW, _D), jnp.float32),
        pltpu.VMEM((_S,), jnp.float32),
        pltpu.SemaphoreType.DMA,
    ],
)
def _emb_kernel(idx_hbm, table_hbm, shared_hbm, out_hbm, idx_v, rows_v, sh_v, sem):
    wid = lax.axis_index("s") * _NC + lax.axis_index("c")
    base = wid * _BPW
    pltpu.sync_copy(idx_hbm.at[pl.ds(base, _BPW)], idx_v)
    gather = pltpu.async_copy(table_hbm.at[idx_v], rows_v, sem)
    pltpu.sync_copy(shared_hbm.at[0], sh_v)
    gather.wait()
    sval = sh_v[...]

    def body(i, carry):
        rows_v[i, pl.ds(0, _S)] = sval
        return carry

    lax.fori_loop(0, _BPW, body, 0, unroll=8)
    pltpu.sync_copy(rows_v, out_hbm.at[pl.ds(base, _BPW)])


def kernel(X, W, shared_embed):
    return _emb_kernel(X.astype(jnp.int32), W, shared_embed)


# speedup vs baseline: 1.7113x; 1.7113x over previous
import functools

import jax
import jax.numpy as jnp
from jax import lax
from jax.experimental import pallas as pl
from jax.experimental.pallas import tpu as pltpu
from jax.experimental.pallas import tpu_sc as plsc

_B = 16384
_D = 64
_S = 16

_info = plsc.get_sparse_core_info()
_NC = _info.num_cores
_NW = _info.num_cores * _info.num_subcores
_BPW = _B // _NW

_mesh = plsc.VectorSubcoreMesh(core_axis_name="c", subcore_axis_name="s")


@functools.partial(
    pl.kernel,
    mesh=_mesh,
    compiler_params=pltpu.CompilerParams(needs_layout_passes=False),
    out_type=jax.ShapeDtypeStruct((_B, _D), jnp.float32),
    scratch_types=[
        pltpu.VMEM((_BPW,), jnp.int32),
        pltpu.VMEM((_BPW, _D), jnp.float32),
        pltpu.VMEM((_S,), jnp.float32),
        pltpu.SemaphoreType.DMA,
    ],
)
def _emb_kernel(idx_hbm, table_hbm, shared_hbm, out_hbm,
                idx_v, rows_v, sh_v, sem):
    wid = lax.axis_index("s") * _NC + lax.axis_index("c")
    base = wid * _BPW
    pltpu.sync_copy(idx_hbm.at[pl.ds(base, _BPW)], idx_v)
    pltpu.sync_copy(shared_hbm.at[0], sh_v)

    lanes = lax.iota(jnp.int32, 16)

    def fire(blk, carry):
        vec = idx_v[pl.ds(blk * 16, 16)]
        for l in range(16):
            r = jnp.sum(jnp.where(lanes == l, vec, 0))
            pltpu.async_copy(table_hbm.at[r], rows_v.at[blk * 16 + l], sem)
        return carry

    lax.fori_loop(0, _BPW // 16, fire, 0, unroll=1)

    def drain(i, carry):
        pltpu.make_async_copy(table_hbm.at[0], rows_v.at[i], sem).wait()
        return carry

    lax.fori_loop(0, _BPW, drain, 0, unroll=8)

    sval = sh_v[...]

    def body(i, carry):
        rows_v[i, pl.ds(0, _S)] = sval
        return carry

    lax.fori_loop(0, _BPW, body, 0, unroll=8)
    pltpu.sync_copy(rows_v, out_hbm.at[pl.ds(base, _BPW)])


def kernel(X, W, shared_embed):
    return _emb_kernel(X.astype(jnp.int32), W, shared_embed)
